# Initial kernel scaffold; baseline (speedup 1.0000x reference)
import jax
import jax.numpy as jnp
from jax.experimental import pallas as pl


def kernel(x, edge_index, edge_type, w1, q1, k1, b1, w2, q2, k2, b2, w3, q3, k3, b3):
    raise NotImplementedError("write your pallas kernel here")


# speedup vs baseline: 36.6476x; 36.6476x over previous
import functools

import jax
import jax.numpy as jnp
from jax import lax
from jax.experimental import pallas as pl
from jax.experimental.pallas import tpu as pltpu
from jax.experimental.pallas import tpu_sc as plsc

N = 10000
E = 320000
R = 6
D1 = 128
H = 64
NCLS = 2
DP3 = 16
NEG = 0.2

NCORE = 2
NTILE = 16
NWORK = NCORE * NTILE
EP = E // NWORK
C = 400
NCH = EP // C
SUB = 80
NSB = C // SUB
NPAD = 10240
PT = NPAD // NTILE

F32 = jnp.float32
I32 = jnp.int32

_MESH = plsc.VectorSubcoreMesh(
    core_axis_name="c", subcore_axis_name="s",
    num_cores=NCORE, num_subcores=NTILE)

_SC_PARAMS = pltpu.CompilerParams(needs_layout_passes=False,
                                  use_tc_tiling_on_sc=False)


BN = 1000
GRID = N // BN

_DN = (((1,), (0,)), ((), ()))


def _mm(a, b):
  return lax.dot_general(a, b, dimension_numbers=_DN,
                         preferred_element_type=F32)


def _tc_layer(din, dout, dpad, first):
  def body(*refs):
    if first:
      x_ref, w_ref, q_ref, k_ref, xrel_ref, sq_ref, sk_ref = refs
      hb = x_ref[...]
    else:
      p_ref, b_ref, w_ref, q_ref, k_ref, xrel_ref, sq_ref, sk_ref = refs
      hb = jnp.maximum(p_ref[0] + p_ref[1] + b_ref[...], 0.0)
    outs, sqs, sks = [], [], []
    for r in range(R):
      hr = _mm(hb, w_ref[r])
      sqs.append(_mm(hr, q_ref[...]))
      sks.append(_mm(hr, k_ref[...]))
      if dpad != dout:
        hr = jnp.concatenate(
            [hr, jnp.zeros((BN, dpad - dout), F32)], axis=1)
      outs.append(hr)
    xrel_ref[...] = jnp.stack(outs, axis=1)
    sq_ref[...] = jnp.concatenate(sqs, axis=1)
    sk_ref[...] = jnp.concatenate(sks, axis=1)

  if first:
    in_specs = [pl.BlockSpec((BN, din), lambda i: (i, 0))]
  else:
    in_specs = [pl.BlockSpec((2, BN, din), lambda i: (0, i, 0)),
                pl.BlockSpec((1, din), lambda i: (0, 0))]
  in_specs += [pl.BlockSpec((R, din, dout), lambda i: (0, 0, 0)),
               pl.BlockSpec((dout, 1), lambda i: (0, 0)),
               pl.BlockSpec((dout, 1), lambda i: (0, 0))]
  return pl.pallas_call(
      body,
      grid=(GRID,),
      in_specs=in_specs,
      out_specs=[pl.BlockSpec((BN, R, dpad), lambda i: (i, 0, 0)),
                 pl.BlockSpec((BN, R), lambda i: (i, 0)),
                 pl.BlockSpec((BN, R), lambda i: (i, 0))],
      out_shape=[jax.ShapeDtypeStruct((N, R, dpad), F32),
                 jax.ShapeDtypeStruct((N, R), F32),
                 jax.ShapeDtypeStruct((N, R), F32)])


def _tc_final(outp3, b3):
  def body(p_ref, b_ref, o_ref):
    x1 = p_ref[0][:, :NCLS] + p_ref[1][:, :NCLS] + b_ref[...]
    m = jnp.max(x1, axis=1, keepdims=True)
    lse = m + jnp.log(jnp.sum(jnp.exp(x1 - m), axis=1, keepdims=True))
    o_ref[...] = x1 - lse

  return pl.pallas_call(
      body,
      grid=(GRID,),
      in_specs=[pl.BlockSpec((2, BN, DP3), lambda i: (0, i, 0)),
                pl.BlockSpec((1, NCLS), lambda i: (0, 0))],
      out_specs=pl.BlockSpec((BN, NCLS), lambda i: (i, 0)),
      out_shape=jax.ShapeDtypeStruct((N, NCLS), F32))(outp3, b3)


def _sc_a_body(sq_hbm, sk_hbm, src_hbm, dst_hbm, et_hbm,
               ex_hbm, den_hbm,
               sq_v, sk_v, src_v, dst_v, et_v, ex_v, dst2_v, den_sh, sem):
  del sem
  cid = lax.axis_index("c")
  sid = lax.axis_index("s")
  wid = cid * NTILE + sid

  def zbody(i, _):
    ex_v[pl.ds(i * 16, 16)] = jnp.zeros((16,), F32)
    return 0
  lax.fori_loop(0, C // 16, zbody, 0)
  pltpu.sync_copy(ex_v, den_sh.at[pl.ds(sid * PT, C)])
  pltpu.sync_copy(ex_v.at[pl.ds(0, PT - C)],
                  den_sh.at[pl.ds(sid * PT + C, PT - C)])
  pltpu.sync_copy(sq_hbm, sq_v)
  pltpu.sync_copy(sk_hbm, sk_v)
  plsc.subcore_barrier()

  def chunk(g, _):
    base = wid * EP + g * C
    pltpu.sync_copy(src_hbm.at[pl.ds(base, C)], src_v)
    pltpu.sync_copy(dst_hbm.at[pl.ds(base, C)], dst_v)
    pltpu.sync_copy(et_hbm.at[pl.ds(base, C)], et_v)
    for v in range(C // 16):
      row, col = v // (SUB // 16), (v % (SUB // 16)) * 16
      sl = pl.ds(v * 16, 16)
      s = src_v[sl]
      t = et_v[sl]
      d = dst_v[sl]
      dst2_v[row, pl.ds(col, 16)] = d
      qi = plsc.load_gather(sq_v, [d * R + t])
      kj = plsc.load_gather(sk_v, [s * R + t])
      a = qi + kj
      a = jnp.maximum(a, a * NEG)
      ex_v[sl] = jnp.exp(a)
    for j in range(NSB):
      pltpu.sync_copy(ex_v.at[pl.ds(j * SUB, SUB)],
                      den_sh.at[dst2_v.at[j]], add=True)
    pltpu.sync_copy(ex_v, ex_hbm.at[pl.ds(base, C)])
    return 0
  lax.fori_loop(0, NCH, chunk, 0)
  plsc.subcore_barrier()
  pltpu.sync_copy(den_sh.at[pl.ds(sid * PT, PT)],
                  den_hbm.at[cid, pl.ds(sid * PT, PT)])


_sc_edge_a = pl.kernel(
    _sc_a_body,
    out_type=(jax.ShapeDtypeStruct((E,), F32),
              jax.ShapeDtypeStruct((NCORE, NPAD), F32)),
    mesh=_MESH,
    compiler_params=_SC_PARAMS,
    scratch_types=[
        pltpu.VMEM((N * R,), F32),
        pltpu.VMEM((N * R,), F32),
        pltpu.VMEM((C,), I32),
        pltpu.VMEM((C,), I32),
        pltpu.VMEM((C,), I32),
        pltpu.VMEM((C,), F32),
        pltpu.VMEM((NSB, SUB), I32),
        pltpu.VMEM_SHARED((NPAD,), F32),
        pltpu.SemaphoreType.DMA,
    ])


def _sc_b_body(dout, xrel_hbm, ex_hbm, den_hbm, src_hbm, dst_hbm, et_hbm,
               outp_hbm,
               den0_v, den1_v, src_v, dst_v, et_v, ex_v, w_v, dst2_v, idx2_v,
               rows_v, out_sh, sem):
  cid = lax.axis_index("c")
  sid = lax.axis_index("s")
  wid = cid * NTILE + sid
  ncb = dout // 16
  nrt = NPAD // NTILE

  def zr(e, _):
    for cc in range(ncb):
      rows_v[e, pl.ds(cc * 16, 16)] = jnp.zeros((16,), F32)
    return 0
  lax.fori_loop(0, C, zr, 0)
  pltpu.sync_copy(rows_v, out_sh.at[pl.ds(sid * nrt, C)])
  pltpu.sync_copy(rows_v.at[pl.ds(0, nrt - C)],
                  out_sh.at[pl.ds(sid * nrt + C, nrt - C)])
  pltpu.sync_copy(den_hbm.at[0], den0_v)
  pltpu.sync_copy(den_hbm.at[1], den1_v)

  def dc(i, _):
    sl = pl.ds(i * 16, 16)
    den0_v[sl] = den0_v[sl] + den1_v[sl]
    return 0
  lax.fori_loop(0, NPAD // 16, dc, 0)
  plsc.subcore_barrier()

  def chunk(g, _):
    base = wid * EP + g * C
    pltpu.sync_copy(src_hbm.at[pl.ds(base, C)], src_v)
    pltpu.sync_copy(dst_hbm.at[pl.ds(base, C)], dst_v)
    pltpu.sync_copy(et_hbm.at[pl.ds(base, C)], et_v)
    pltpu.sync_copy(ex_hbm.at[pl.ds(base, C)], ex_v)
    for v in range(C // 16):
      row, col = v // (SUB // 16), (v % (SUB // 16)) * 16
      sl = pl.ds(v * 16, 16)
      s = src_v[sl]
      t = et_v[sl]
      d = dst_v[sl]
      dst2_v[row, pl.ds(col, 16)] = d
      den = plsc.load_gather(den0_v, [d])
      w_v[sl] = ex_v[sl] / (den + 1e-16)
      idx2_v[row, pl.ds(col, 16)] = s * R + t
    cps = [pltpu.async_copy(xrel_hbm.at[idx2_v.at[j]],
                            rows_v.at[pl.ds(j * SUB, SUB)], sem)
           for j in range(NSB)]
    for cp in cps:
      cp.wait()

    def scale(b, _):
      wv = w_v[pl.ds(b * 16, 16)]
      for j in range(16):
        e = b * 16 + j
        wj = jnp.broadcast_to(wv[j], (16,))
        for cc in range(ncb):
          sl2 = pl.ds(cc * 16, 16)
          rows_v[e, sl2] = rows_v[e, sl2] * wj
      return 0
    lax.fori_loop(0, C // 16, scale, 0)
    for j in range(NSB):
      pltpu.sync_copy(rows_v.at[pl.ds(j * SUB, SUB)],
                      out_sh.at[dst2_v.at[j]], add=True)
    return 0
  lax.fori_loop(0, NCH, chunk, 0)
  plsc.subcore_barrier()
  pltpu.sync_copy(out_sh.at[pl.ds(sid * nrt, nrt)],
                  outp_hbm.at[cid, pl.ds(sid * nrt, nrt)])


def _make_sc_b(dout):
  return pl.kernel(
      functools.partial(_sc_b_body, dout),
      out_type=jax.ShapeDtypeStruct((NCORE, NPAD, dout), F32),
      mesh=_MESH,
      compiler_params=_SC_PARAMS,
      scratch_types=[
          pltpu.VMEM((NPAD,), F32),
          pltpu.VMEM((NPAD,), F32),
          pltpu.VMEM((C,), I32),
          pltpu.VMEM((C,), I32),
          pltpu.VMEM((C,), I32),
          pltpu.VMEM((C,), F32),
          pltpu.VMEM((C,), F32),
          pltpu.VMEM((NSB, SUB), I32),
          pltpu.VMEM((NSB, SUB), I32),
          pltpu.VMEM((C, dout), F32),
          pltpu.VMEM_SHARED((NPAD, dout), F32),
          pltpu.SemaphoreType.DMA,
      ])


_sc_edge_b64 = _make_sc_b(H)
_sc_edge_b16 = _make_sc_b(DP3)


def kernel(x, edge_index, edge_type, w1, q1, k1, b1, w2, q2, k2, b2,
           w3, q3, k3, b3):
  src = edge_index[0].astype(I32)
  dst = edge_index[1].astype(I32)
  et = edge_type.astype(I32)

  def edge_phase(xrel, sq, sk, sc_b, dpad):
    ex, den = _sc_edge_a(sq.reshape(N * R), sk.reshape(N * R),
                         src, dst, et)
    return sc_b(xrel.reshape(N * R, dpad), ex, den, src, dst, et)

  xrel, sq, sk = _tc_layer(D1, H, H, True)(x, w1, q1, k1)
  outp = edge_phase(xrel, sq, sk, _sc_edge_b64, H)

  xrel, sq, sk = _tc_layer(H, H, H, False)(
      outp, b1.reshape(1, H), w2, q2, k2)
  outp = edge_phase(xrel, sq, sk, _sc_edge_b64, H)

  xrel, sq, sk = _tc_layer(H, NCLS, DP3, False)(
      outp, b2.reshape(1, H), w3, q3, k3)
  outp = edge_phase(xrel, sq, sk, _sc_edge_b16, DP3)

  return _tc_final(outp, b3.reshape(1, NCLS))


# speedup vs baseline: 44.5363x; 1.2153x over previous
import functools

import jax
import jax.numpy as jnp
from jax import lax
from jax.experimental import pallas as pl
from jax.experimental.pallas import tpu as pltpu
from jax.experimental.pallas import tpu_sc as plsc

N = 10000
E = 320000
R = 6
D1 = 128
H = 64
NCLS = 2
DP3 = 16
NEG = 0.2

NCORE = 2
NTILE = 16
NWORK = NCORE * NTILE
EP = E // NWORK
C = 400
NCH = EP // C
SUB = 80
NSB = C // SUB
NPAD = 10240
PT = NPAD // NTILE

F32 = jnp.float32
I32 = jnp.int32

_MESH = plsc.VectorSubcoreMesh(
    core_axis_name="c", subcore_axis_name="s",
    num_cores=NCORE, num_subcores=NTILE)

_SC_PARAMS = pltpu.CompilerParams(needs_layout_passes=False,
                                  use_tc_tiling_on_sc=False)


BN = 1000
GRID = N // BN

_DN = (((1,), (0,)), ((), ()))


def _mm(a, b):
  return lax.dot_general(a, b, dimension_numbers=_DN,
                         preferred_element_type=F32)


def _tc_layer(din, dout, dpad, first):
  def body(*refs):
    if first:
      x_ref, w_ref, q_ref, k_ref, xrel_ref, sq_ref, sk_ref = refs
      hb = x_ref[...]
    else:
      p_ref, b_ref, w_ref, q_ref, k_ref, xrel_ref, sq_ref, sk_ref = refs
      hb = jnp.maximum(p_ref[0] + p_ref[1] + b_ref[...], 0.0)
    outs, sqs, sks = [], [], []
    for r in range(R):
      hr = _mm(hb, w_ref[r])
      sqs.append(_mm(hr, q_ref[...]))
      sks.append(_mm(hr, k_ref[...]))
      if dpad != dout:
        hr = jnp.concatenate(
            [hr, jnp.zeros((BN, dpad - dout), F32)], axis=1)
      outs.append(hr)
    xrel_ref[...] = jnp.stack(outs, axis=1)
    sq_ref[...] = jnp.concatenate(sqs, axis=1)
    sk_ref[...] = jnp.concatenate(sks, axis=1)

  if first:
    in_specs = [pl.BlockSpec((BN, din), lambda i: (i, 0))]
  else:
    in_specs = [pl.BlockSpec((2, BN, din), lambda i: (0, i, 0)),
                pl.BlockSpec((1, din), lambda i: (0, 0))]
  in_specs += [pl.BlockSpec((R, din, dout), lambda i: (0, 0, 0)),
               pl.BlockSpec((dout, 1), lambda i: (0, 0)),
               pl.BlockSpec((dout, 1), lambda i: (0, 0))]
  return pl.pallas_call(
      body,
      grid=(GRID,),
      in_specs=in_specs,
      out_specs=[pl.BlockSpec((BN, R, dpad), lambda i: (i, 0, 0)),
                 pl.BlockSpec((BN, R), lambda i: (i, 0)),
                 pl.BlockSpec((BN, R), lambda i: (i, 0))],
      out_shape=[jax.ShapeDtypeStruct((N, R, dpad), F32),
                 jax.ShapeDtypeStruct((N, R), F32),
                 jax.ShapeDtypeStruct((N, R), F32)])


def _tc_final(outp3, b3):
  def body(p_ref, b_ref, o_ref):
    x1 = p_ref[0][:, :NCLS] + p_ref[1][:, :NCLS] + b_ref[...]
    m = jnp.max(x1, axis=1, keepdims=True)
    lse = m + jnp.log(jnp.sum(jnp.exp(x1 - m), axis=1, keepdims=True))
    o_ref[...] = x1 - lse

  return pl.pallas_call(
      body,
      grid=(GRID,),
      in_specs=[pl.BlockSpec((2, BN, DP3), lambda i: (0, i, 0)),
                pl.BlockSpec((1, NCLS), lambda i: (0, 0))],
      out_specs=pl.BlockSpec((BN, NCLS), lambda i: (i, 0)),
      out_shape=jax.ShapeDtypeStruct((N, NCLS), F32))(outp3, b3)


NVC = C // 16
NRC = C // SUB


def _sc_a_body(sq_hbm, sk_hbm, dstet2_hbm, srk2_hbm, dst2_hbm,
               ex_hbm, den_hbm,
               sq_v, sk_v, dstet_v, srk_v, dst2_v, ex_v, den_sh, sem):
  cid = lax.axis_index("c")
  sid = lax.axis_index("s")
  wid = cid * NTILE + sid

  def zbody(i, _):
    ex_v[pl.ds(i * 16, 16)] = jnp.zeros((16,), F32)
    return 0
  lax.fori_loop(0, PT // 16, zbody, 0)
  pltpu.sync_copy(ex_v.at[pl.ds(0, PT)], den_sh.at[pl.ds(sid * PT, PT)])
  pltpu.sync_copy(sq_hbm, sq_v)
  pltpu.sync_copy(sk_hbm, sk_v)
  plsc.subcore_barrier()

  def chunk(g, _):
    base = wid * EP + g * C
    rbase = wid * (EP // SUB) + g * NRC
    pltpu.sync_copy(dstet2_hbm.at[pl.ds(rbase, NRC)], dstet_v)
    pltpu.sync_copy(srk2_hbm.at[pl.ds(rbase, NRC)], srk_v)
    pltpu.sync_copy(dst2_hbm.at[pl.ds(rbase, NRC)], dst2_v)
    for v in range(NVC):
      row, col = v // (SUB // 16), (v % (SUB // 16)) * 16
      csl = pl.ds(col, 16)
      qi = plsc.load_gather(sq_v, [dstet_v[row, csl]])
      kj = plsc.load_gather(sk_v, [srk_v[row, csl]])
      a = qi + kj
      a = jnp.maximum(a, a * NEG)
      ex_v[pl.ds(v * 16, 16)] = jnp.exp(a)
    for j in range(NRC):
      pltpu.async_copy(ex_v.at[pl.ds(j * SUB, SUB)],
                       den_sh.at[dst2_v.at[j]], sem, add=True)
    pltpu.sync_copy(ex_v.at[pl.ds(0, C)], ex_hbm.at[pl.ds(base, C)])
    pltpu.make_async_copy(ex_hbm.at[pl.ds(base, C)],
                          ex_v.at[pl.ds(0, C)], sem).wait()
    return 0
  lax.fori_loop(0, NCH, chunk, 0)
  plsc.subcore_barrier()
  pltpu.sync_copy(den_sh.at[pl.ds(sid * PT, PT)],
                  den_hbm.at[cid, pl.ds(sid * PT, PT)])


_sc_edge_a = pl.kernel(
    _sc_a_body,
    out_type=(jax.ShapeDtypeStruct((E,), F32),
              jax.ShapeDtypeStruct((NCORE, NPAD), F32)),
    mesh=_MESH,
    compiler_params=_SC_PARAMS,
    scratch_types=[
        pltpu.VMEM((N * R,), F32),
        pltpu.VMEM((N * R,), F32),
        pltpu.VMEM((NRC, SUB), I32),
        pltpu.VMEM((NRC, SUB), I32),
        pltpu.VMEM((NRC, SUB), I32),
        pltpu.VMEM((PT,), F32),
        pltpu.VMEM_SHARED((NPAD,), F32),
        pltpu.SemaphoreType.DMA,
    ])


GB = 2000
NGB = EP // GB
NCG = GB // C
NRG = GB // SUB


def _sc_b_body(dout, xrel_hbm, ex_hbm, den_hbm, srk2_hbm, dst2_hbm,
               outp_hbm,
               den0_v, den1_v, srk_v, dst2_v, ex_v,
               w_v0, w_v1, rows_v0, rows_v1, out_sh, sem0, sem1):
  cid = lax.axis_index("c")
  sid = lax.axis_index("s")
  wid = cid * NTILE + sid
  ncb = dout // 16
  nrt = NPAD // NTILE

  def zr(e, _):
    for cc in range(ncb):
      rows_v0[e, pl.ds(cc * 16, 16)] = jnp.zeros((16,), F32)
    return 0
  lax.fori_loop(0, C, zr, 0)
  pltpu.sync_copy(rows_v0, out_sh.at[pl.ds(sid * nrt, C)])
  pltpu.sync_copy(rows_v0.at[pl.ds(0, nrt - C)],
                  out_sh.at[pl.ds(sid * nrt + C, nrt - C)])
  pltpu.sync_copy(den_hbm.at[0], den0_v)
  pltpu.sync_copy(den_hbm.at[1], den1_v)

  def dc(i, _):
    sl = pl.ds(i * 16, 16)
    den0_v[sl] = den0_v[sl] + den1_v[sl]
    return 0
  lax.fori_loop(0, NPAD // 16, dc, 0)
  plsc.subcore_barrier()

  bufs = [(w_v0, rows_v0, sem0), (w_v1, rows_v1, sem1)]

  def fire(c, buf):
    w_v, rows_v, sem = bufs[buf]
    for v in range(C // 16):
      gv = c * (C // 16) + v
      row, col = gv // (SUB // 16), (gv % (SUB // 16)) * 16
      d = dst2_v[row, pl.ds(col, 16)]
      den = plsc.load_gather(den0_v, [d])
      w_v[pl.ds(v * 16, 16)] = ex_v[pl.ds(c * C + v * 16, 16)] / (den + 1e-16)
    for j in range(NSB):
      pltpu.async_copy(xrel_hbm.at[srk_v.at[c * NSB + j]],
                       rows_v.at[pl.ds(j * SUB, SUB)], sem)

  def process(c, buf):
    w_v, rows_v, sem = bufs[buf]
    pltpu.make_async_copy(xrel_hbm.at[pl.ds(0, C)], rows_v, sem).wait()

    def scale(b, _):
      wv = w_v[pl.ds(b * 16, 16)]
      for j in range(16):
        e = b * 16 + j
        wj = jnp.broadcast_to(wv[j], (16,))
        for cc in range(ncb):
          sl2 = pl.ds(cc * 16, 16)
          rows_v[e, sl2] = rows_v[e, sl2] * wj
      return 0
    lax.fori_loop(0, C // 16, scale, 0)
    for j in range(NSB):
      pltpu.sync_copy(rows_v.at[pl.ds(j * SUB, SUB)],
                      out_sh.at[dst2_v.at[c * NSB + j]], add=True)

  def group(g, _):
    rbase = wid * (EP // SUB) + g * NRG
    pltpu.sync_copy(srk2_hbm.at[pl.ds(rbase, NRG)], srk_v)
    pltpu.sync_copy(dst2_hbm.at[pl.ds(rbase, NRG)], dst2_v)
    pltpu.sync_copy(ex_hbm.at[pl.ds(wid * EP + g * GB, GB)], ex_v)
    fire(0, 0)
    for c in range(1, NCG):
      fire(c, c % 2)
      process(c - 1, (c - 1) % 2)
    process(NCG - 1, (NCG - 1) % 2)
    return 0
  lax.fori_loop(0, NGB, group, 0)

  plsc.subcore_barrier()
  pltpu.sync_copy(out_sh.at[pl.ds(sid * nrt, nrt)],
                  outp_hbm.at[cid, pl.ds(sid * nrt, nrt)])


def _make_sc_b(dout):
  return pl.kernel(
      functools.partial(_sc_b_body, dout),
      out_type=jax.ShapeDtypeStruct((NCORE, NPAD, dout), F32),
      mesh=_MESH,
      compiler_params=_SC_PARAMS,
      scratch_types=[
          pltpu.VMEM((NPAD,), F32),
          pltpu.VMEM((NPAD,), F32),
          pltpu.VMEM((NRG, SUB), I32),
          pltpu.VMEM((NRG, SUB), I32),
          pltpu.VMEM((GB,), F32),
          pltpu.VMEM((C,), F32),
          pltpu.VMEM((C,), F32),
          pltpu.VMEM((C, dout), F32),
          pltpu.VMEM((C, dout), F32),
          pltpu.VMEM_SHARED((NPAD, dout), F32),
          pltpu.SemaphoreType.DMA,
          pltpu.SemaphoreType.DMA,
      ])


_sc_edge_b64 = _make_sc_b(H)
_sc_edge_b16 = _make_sc_b(DP3)


def kernel(x, edge_index, edge_type, w1, q1, k1, b1, w2, q2, k2, b2,
           w3, q3, k3, b3):
  src = edge_index[0].astype(I32)
  dst = edge_index[1].astype(I32)
  et = edge_type.astype(I32)
  srk2 = (src * R + et).reshape(E // SUB, SUB)
  dstet2 = (dst * R + et).reshape(E // SUB, SUB)
  dst2 = dst.reshape(E // SUB, SUB)

  def edge_phase(xrel, sq, sk, sc_b, dpad):
    ex, den = _sc_edge_a(sq.reshape(N * R), sk.reshape(N * R),
                         dstet2, srk2, dst2)
    return sc_b(xrel.reshape(N * R, dpad), ex, den, srk2, dst2)

  xrel, sq, sk = _tc_layer(D1, H, H, True)(x, w1, q1, k1)
  outp = edge_phase(xrel, sq, sk, _sc_edge_b64, H)

  xrel, sq, sk = _tc_layer(H, H, H, False)(
      outp, b1.reshape(1, H), w2, q2, k2)
  outp = edge_phase(xrel, sq, sk, _sc_edge_b64, H)

  xrel, sq, sk = _tc_layer(H, NCLS, DP3, False)(
      outp, b2.reshape(1, H), w3, q3, k3)
  outp = edge_phase(xrel, sq, sk, _sc_edge_b16, DP3)

  return _tc_final(outp, b3.reshape(1, NCLS))


# speedup vs baseline: 49.1216x; 1.1030x over previous
import functools

import jax
import jax.numpy as jnp
from jax import lax
from jax.experimental import pallas as pl
from jax.experimental.pallas import tpu as pltpu
from jax.experimental.pallas import tpu_sc as plsc

N = 10000
E = 320000
R = 6
D1 = 128
H = 64
NCLS = 2
DP3 = 16
NEG = 0.2

NCORE = 2
NTILE = 16
NWORK = NCORE * NTILE
EP = E // NWORK
C = 400
NCH = EP // C
SUB = 80
NSB = C // SUB
NPAD = 10240
PT = NPAD // NTILE

F32 = jnp.float32
I32 = jnp.int32

_MESH = plsc.VectorSubcoreMesh(
    core_axis_name="c", subcore_axis_name="s",
    num_cores=NCORE, num_subcores=NTILE)

_SC_PARAMS = pltpu.CompilerParams(needs_layout_passes=False,
                                  use_tc_tiling_on_sc=False)


BN = 1000
GRID = N // BN

_DN = (((1,), (0,)), ((), ()))


def _mm(a, b):
  return lax.dot_general(a, b, dimension_numbers=_DN,
                         preferred_element_type=F32)


def _tc_layer(din, dout, dpad, first):
  def body(*refs):
    if first:
      x_ref, w_ref, q_ref, k_ref, xrel_ref, sq_ref, sk_ref = refs
      hb = x_ref[...]
    else:
      p_ref, b_ref, w_ref, q_ref, k_ref, xrel_ref, sq_ref, sk_ref = refs
      hb = jnp.maximum(p_ref[0] + p_ref[1] + b_ref[...], 0.0)
    outs, sqs, sks = [], [], []
    for r in range(R):
      hr = _mm(hb, w_ref[r])
      sqs.append(_mm(hr, q_ref[...]))
      sks.append(_mm(hr, k_ref[...]))
      if dpad != dout:
        hr = jnp.concatenate(
            [hr, jnp.zeros((BN, dpad - dout), F32)], axis=1)
      outs.append(hr)
    xrel_ref[...] = jnp.stack(outs, axis=1)
    sq_ref[...] = jnp.concatenate(sqs, axis=1)
    sk_ref[...] = jnp.concatenate(sks, axis=1)

  if first:
    in_specs = [pl.BlockSpec((BN, din), lambda i: (i, 0))]
  else:
    in_specs = [pl.BlockSpec((2, BN, din), lambda i: (0, i, 0)),
                pl.BlockSpec((1, din), lambda i: (0, 0))]
  in_specs += [pl.BlockSpec((R, din, dout), lambda i: (0, 0, 0)),
               pl.BlockSpec((dout, 1), lambda i: (0, 0)),
               pl.BlockSpec((dout, 1), lambda i: (0, 0))]
  return pl.pallas_call(
      body,
      grid=(GRID,),
      in_specs=in_specs,
      out_specs=[pl.BlockSpec((BN, R, dpad), lambda i: (i, 0, 0)),
                 pl.BlockSpec((BN, R), lambda i: (i, 0)),
                 pl.BlockSpec((BN, R), lambda i: (i, 0))],
      out_shape=[jax.ShapeDtypeStruct((N, R, dpad), F32),
                 jax.ShapeDtypeStruct((N, R), F32),
                 jax.ShapeDtypeStruct((N, R), F32)])


def _tc_final(outp3, b3):
  def body(p_ref, b_ref, o_ref):
    x1 = p_ref[0][:, :NCLS] + p_ref[1][:, :NCLS] + b_ref[...]
    m = jnp.max(x1, axis=1, keepdims=True)
    lse = m + jnp.log(jnp.sum(jnp.exp(x1 - m), axis=1, keepdims=True))
    o_ref[...] = x1 - lse

  return pl.pallas_call(
      body,
      grid=(GRID,),
      in_specs=[pl.BlockSpec((2, BN, DP3), lambda i: (0, i, 0)),
                pl.BlockSpec((1, NCLS), lambda i: (0, 0))],
      out_specs=pl.BlockSpec((BN, NCLS), lambda i: (i, 0)),
      out_shape=jax.ShapeDtypeStruct((N, NCLS), F32))(outp3, b3)


NVC = C // 16
NRC = C // SUB


def _sc_a_body(sq_hbm, sk_hbm, dstet2_hbm, srk2_hbm, dst2_hbm,
               ex_hbm, den_hbm,
               sq_v, sk_v, dstet_v0, dstet_v1, srk_v0, srk_v1,
               dst2_v0, dst2_v1, ex_v0, ex_v1, den_sh,
               seml0, seml1, sems0, sems1):
  cid = lax.axis_index("c")
  sid = lax.axis_index("s")
  wid = cid * NTILE + sid

  def zbody(i, _):
    ex_v0[pl.ds(i * 16, 16)] = jnp.zeros((16,), F32)
    return 0
  lax.fori_loop(0, PT // 16, zbody, 0)
  pltpu.sync_copy(ex_v0.at[pl.ds(0, PT)], den_sh.at[pl.ds(sid * PT, PT)])
  pltpu.sync_copy(sq_hbm, sq_v)
  pltpu.sync_copy(sk_hbm, sk_v)
  plsc.subcore_barrier()

  bufs = [(dstet_v0, srk_v0, dst2_v0, ex_v0, seml0, sems0),
          (dstet_v1, srk_v1, dst2_v1, ex_v1, seml1, sems1)]

  def loadf(g, buf):
    dstet_v, srk_v, dst2_v, _, seml, _ = bufs[buf]
    rbase = wid * (EP // SUB) + g * NRC
    pltpu.async_copy(dstet2_hbm.at[pl.ds(rbase, NRC)], dstet_v, seml)
    pltpu.async_copy(srk2_hbm.at[pl.ds(rbase, NRC)], srk_v, seml)
    pltpu.async_copy(dst2_hbm.at[pl.ds(rbase, NRC)], dst2_v, seml)

  def compute(g, buf):
    dstet_v, srk_v, dst2_v, ex_v, seml, sems = bufs[buf]
    rbase = wid * (EP // SUB)
    pltpu.make_async_copy(dstet2_hbm.at[pl.ds(rbase, NRC)], dstet_v,
                          seml).wait()
    pltpu.make_async_copy(srk2_hbm.at[pl.ds(rbase, NRC)], srk_v,
                          seml).wait()
    pltpu.make_async_copy(dst2_hbm.at[pl.ds(rbase, NRC)], dst2_v,
                          seml).wait()
    for v in range(NVC):
      row, col = v // (SUB // 16), (v % (SUB // 16)) * 16
      csl = pl.ds(col, 16)
      qi = plsc.load_gather(sq_v, [dstet_v[row, csl]])
      kj = plsc.load_gather(sk_v, [srk_v[row, csl]])
      a = qi + kj
      a = jnp.maximum(a, a * NEG)
      ex_v[pl.ds(v * 16, 16)] = jnp.exp(a)
    del sems
    for j in range(NRC):
      pltpu.sync_copy(ex_v.at[pl.ds(j * SUB, SUB)],
                      den_sh.at[dst2_v.at[j]], add=True)
    base = wid * EP + g * C
    pltpu.sync_copy(ex_v.at[pl.ds(0, C)], ex_hbm.at[pl.ds(base, C)])

  loadf(0, 0)

  def pair(h, _):
    loadf(2 * h + 1, 1)
    compute(2 * h, 0)
    loadf(2 * h + 2, 0)
    compute(2 * h + 1, 1)
    return 0
  lax.fori_loop(0, (NCH - 1) // 2, pair, 0)
  compute(NCH - 1, 0)
  plsc.subcore_barrier()
  pltpu.sync_copy(den_sh.at[pl.ds(sid * PT, PT)],
                  den_hbm.at[cid, pl.ds(sid * PT, PT)])


_sc_edge_a = pl.kernel(
    _sc_a_body,
    out_type=(jax.ShapeDtypeStruct((E,), F32),
              jax.ShapeDtypeStruct((NCORE, NPAD), F32)),
    mesh=_MESH,
    compiler_params=_SC_PARAMS,
    scratch_types=[
        pltpu.VMEM((N * R,), F32),
        pltpu.VMEM((N * R,), F32),
        pltpu.VMEM((NRC, SUB), I32),
        pltpu.VMEM((NRC, SUB), I32),
        pltpu.VMEM((NRC, SUB), I32),
        pltpu.VMEM((NRC, SUB), I32),
        pltpu.VMEM((NRC, SUB), I32),
        pltpu.VMEM((NRC, SUB), I32),
        pltpu.VMEM((PT,), F32),
        pltpu.VMEM((PT,), F32),
        pltpu.VMEM_SHARED((NPAD,), F32),
        pltpu.SemaphoreType.DMA,
        pltpu.SemaphoreType.DMA,
        pltpu.SemaphoreType.DMA,
        pltpu.SemaphoreType.DMA,
    ])


GB = 2000
NGB = EP // GB
NCG = GB // C
NRG = GB // SUB


def _sc_b_body(dout, xrel_hbm, ex_hbm, den_hbm, srk2_hbm, dst2_hbm,
               outp_hbm,
               den0_v, den1_v, srk_v, dst2_v, ex_v,
               w_v0, w_v1, rows_v0, rows_v1, out_sh,
               semg0, semg1, sems0, sems1):
  cid = lax.axis_index("c")
  sid = lax.axis_index("s")
  wid = cid * NTILE + sid
  ncb = dout // 16
  nrt = NPAD // NTILE

  def zr(e, _):
    for cc in range(ncb):
      rows_v0[e, pl.ds(cc * 16, 16)] = jnp.zeros((16,), F32)
    return 0
  lax.fori_loop(0, C, zr, 0)
  pltpu.sync_copy(rows_v0, out_sh.at[pl.ds(sid * nrt, C)])
  pltpu.sync_copy(rows_v0.at[pl.ds(0, nrt - C)],
                  out_sh.at[pl.ds(sid * nrt + C, nrt - C)])
  pltpu.sync_copy(den_hbm.at[0], den0_v)
  pltpu.sync_copy(den_hbm.at[1], den1_v)

  def dc(i, _):
    sl = pl.ds(i * 16, 16)
    den0_v[sl] = den0_v[sl] + den1_v[sl]
    return 0
  lax.fori_loop(0, NPAD // 16, dc, 0)
  plsc.subcore_barrier()

  bufs = [(w_v0, rows_v0, semg0, sems0), (w_v1, rows_v1, semg1, sems1)]

  def fire(c, buf):
    w_v, rows_v, semg, _ = bufs[buf]
    for v in range(C // 16):
      gv = c * (C // 16) + v
      row, col = gv // (SUB // 16), (gv % (SUB // 16)) * 16
      d = dst2_v[row, pl.ds(col, 16)]
      den = plsc.load_gather(den0_v, [d])
      w_v[pl.ds(v * 16, 16)] = ex_v[pl.ds(c * C + v * 16, 16)] / (den + 1e-16)
    for j in range(NSB):
      pltpu.async_copy(xrel_hbm.at[srk_v.at[c * NSB + j]],
                       rows_v.at[pl.ds(j * SUB, SUB)], semg)

  def process(c, buf):
    w_v, rows_v, semg, sems = bufs[buf]
    pltpu.make_async_copy(xrel_hbm.at[pl.ds(0, C)], rows_v, semg).wait()

    def scale(b, _):
      wv = w_v[pl.ds(b * 16, 16)]
      for j in range(16):
        e = b * 16 + j
        wj = jnp.broadcast_to(wv[j], (16,))
        for cc in range(ncb):
          sl2 = pl.ds(cc * 16, 16)
          rows_v[e, sl2] = rows_v[e, sl2] * wj
      return 0
    lax.fori_loop(0, C // 16, scale, 0)
    del sems
    for j in range(NSB):
      pltpu.sync_copy(rows_v.at[pl.ds(j * SUB, SUB)],
                      out_sh.at[dst2_v.at[c * NSB + j]], add=True)

  def group(g, _):
    rbase = wid * (EP // SUB) + g * NRG
    pltpu.sync_copy(srk2_hbm.at[pl.ds(rbase, NRG)], srk_v)
    pltpu.sync_copy(dst2_hbm.at[pl.ds(rbase, NRG)], dst2_v)
    pltpu.sync_copy(ex_hbm.at[pl.ds(wid * EP + g * GB, GB)], ex_v)
    fire(0, 0)
    for c in range(1, NCG):
      fire(c, c % 2)
      process(c - 1, (c - 1) % 2)
    process(NCG - 1, (NCG - 1) % 2)
    return 0
  lax.fori_loop(0, NGB, group, 0)

  plsc.subcore_barrier()
  pltpu.sync_copy(out_sh.at[pl.ds(sid * nrt, nrt)],
                  outp_hbm.at[cid, pl.ds(sid * nrt, nrt)])


def _make_sc_b(dout):
  return pl.kernel(
      functools.partial(_sc_b_body, dout),
      out_type=jax.ShapeDtypeStruct((NCORE, NPAD, dout), F32),
      mesh=_MESH,
      compiler_params=_SC_PARAMS,
      scratch_types=[
          pltpu.VMEM((NPAD,), F32),
          pltpu.VMEM((NPAD,), F32),
          pltpu.VMEM((NRG, SUB), I32),
          pltpu.VMEM((NRG, SUB), I32),
          pltpu.VMEM((GB,), F32),
          pltpu.VMEM((C,), F32),
          pltpu.VMEM((C,), F32),
          pltpu.VMEM((C, dout), F32),
          pltpu.VMEM((C, dout), F32),
          pltpu.VMEM_SHARED((NPAD, dout), F32),
          pltpu.SemaphoreType.DMA,
          pltpu.SemaphoreType.DMA,
          pltpu.SemaphoreType.DMA,
          pltpu.SemaphoreType.DMA,
      ])


_sc_edge_b64 = _make_sc_b(H)
_sc_edge_b16 = _make_sc_b(DP3)


def kernel(x, edge_index, edge_type, w1, q1, k1, b1, w2, q2, k2, b2,
           w3, q3, k3, b3):
  src = edge_index[0].astype(I32)
  dst = edge_index[1].astype(I32)
  et = edge_type.astype(I32)
  srk2 = (src * R + et).reshape(E // SUB, SUB)
  dstet2 = (dst * R + et).reshape(E // SUB, SUB)
  dst2 = dst.reshape(E // SUB, SUB)

  def edge_phase(xrel, sq, sk, sc_b, dpad):
    ex, den = _sc_edge_a(sq.reshape(N * R), sk.reshape(N * R),
                         dstet2, srk2, dst2)
    return sc_b(xrel.reshape(N * R, dpad), ex, den, srk2, dst2)

  xrel, sq, sk = _tc_layer(D1, H, H, True)(x, w1, q1, k1)
  outp = edge_phase(xrel, sq, sk, _sc_edge_b64, H)

  xrel, sq, sk = _tc_layer(H, H, H, False)(
      outp, b1.reshape(1, H), w2, q2, k2)
  outp = edge_phase(xrel, sq, sk, _sc_edge_b64, H)

  xrel, sq, sk = _tc_layer(H, NCLS, DP3, False)(
      outp, b2.reshape(1, H), w3, q3, k3)
  outp = edge_phase(xrel, sq, sk, _sc_edge_b16, DP3)

  return _tc_final(outp, b3.reshape(1, NCLS))


# speedup vs baseline: 50.6150x; 1.0304x over previous
import functools

import jax
import jax.numpy as jnp
from jax import lax
from jax.experimental import pallas as pl
from jax.experimental.pallas import tpu as pltpu
from jax.experimental.pallas import tpu_sc as plsc

N = 10000
E = 320000
R = 6
D1 = 128
H = 64
NCLS = 2
DP3 = 16
NEG = 0.2

NCORE = 2
NTILE = 16
NWORK = NCORE * NTILE
EP = E // NWORK
C = 400
NCH = EP // C
SUB = 80
NSB = C // SUB
NPAD = 10240
PT = NPAD // NTILE

F32 = jnp.float32
I32 = jnp.int32

_MESH = plsc.VectorSubcoreMesh(
    core_axis_name="c", subcore_axis_name="s",
    num_cores=NCORE, num_subcores=NTILE)

_SC_PARAMS = pltpu.CompilerParams(needs_layout_passes=False,
                                  use_tc_tiling_on_sc=False)


BN = 1000
GRID = N // BN

_DN = (((1,), (0,)), ((), ()))


def _mm(a, b):
  return lax.dot_general(a, b, dimension_numbers=_DN,
                         preferred_element_type=F32)


def _tc_layer(din, dout, dpad, first):
  def body(*refs):
    if first:
      x_ref, w_ref, q_ref, k_ref, xrel_ref, sq_ref, sk_ref = refs
      hb = x_ref[...]
    else:
      p_ref, b_ref, w_ref, q_ref, k_ref, xrel_ref, sq_ref, sk_ref = refs
      hb = jnp.maximum(p_ref[0] + p_ref[1] + b_ref[...], 0.0)
    outs, sqs, sks = [], [], []
    for r in range(R):
      hr = _mm(hb, w_ref[r])
      sqs.append(_mm(hr, q_ref[...]))
      sks.append(_mm(hr, k_ref[...]))
      if dpad != dout:
        hr = jnp.concatenate(
            [hr, jnp.zeros((BN, dpad - dout), F32)], axis=1)
      outs.append(hr)
    xrel_ref[...] = jnp.stack(outs, axis=1)
    sq_ref[...] = jnp.concatenate(sqs, axis=1)
    sk_ref[...] = jnp.concatenate(sks, axis=1)

  if first:
    in_specs = [pl.BlockSpec((BN, din), lambda i: (i, 0))]
  else:
    in_specs = [pl.BlockSpec((2, BN, din), lambda i: (0, i, 0)),
                pl.BlockSpec((1, din), lambda i: (0, 0))]
  in_specs += [pl.BlockSpec((R, din, dout), lambda i: (0, 0, 0)),
               pl.BlockSpec((dout, 1), lambda i: (0, 0)),
               pl.BlockSpec((dout, 1), lambda i: (0, 0))]
  return pl.pallas_call(
      body,
      grid=(GRID,),
      in_specs=in_specs,
      out_specs=[pl.BlockSpec((BN, R, dpad), lambda i: (i, 0, 0)),
                 pl.BlockSpec((BN, R), lambda i: (i, 0)),
                 pl.BlockSpec((BN, R), lambda i: (i, 0))],
      out_shape=[jax.ShapeDtypeStruct((N, R, dpad), F32),
                 jax.ShapeDtypeStruct((N, R), F32),
                 jax.ShapeDtypeStruct((N, R), F32)])


def _tc_final(outp3, b3):
  def body(p_ref, b_ref, o_ref):
    x1 = p_ref[0][:, :NCLS] + p_ref[1][:, :NCLS] + b_ref[...]
    m = jnp.max(x1, axis=1, keepdims=True)
    lse = m + jnp.log(jnp.sum(jnp.exp(x1 - m), axis=1, keepdims=True))
    o_ref[...] = x1 - lse

  return pl.pallas_call(
      body,
      grid=(GRID,),
      in_specs=[pl.BlockSpec((2, BN, DP3), lambda i: (0, i, 0)),
                pl.BlockSpec((1, NCLS), lambda i: (0, 0))],
      out_specs=pl.BlockSpec((BN, NCLS), lambda i: (i, 0)),
      out_shape=jax.ShapeDtypeStruct((N, NCLS), F32))(outp3, b3)


NVC = C // 16
NRC = C // SUB


def _sc_a_body(sq_hbm, sk_hbm, dstet2_hbm, srk2_hbm, dst2_hbm,
               ex_hbm, den_hbm,
               sq_v, sk_v, dstet_v0, dstet_v1, srk_v0, srk_v1,
               dst2_v0, dst2_v1, ex_v0, ex_v1, den_sh,
               seml0, seml1, sems0, sems1):
  cid = lax.axis_index("c")
  sid = lax.axis_index("s")
  wid = cid * NTILE + sid

  def zbody(i, _):
    ex_v0[pl.ds(i * 16, 16)] = jnp.zeros((16,), F32)
    return 0
  lax.fori_loop(0, PT // 16, zbody, 0)
  pltpu.sync_copy(ex_v0.at[pl.ds(0, PT)], den_sh.at[pl.ds(sid * PT, PT)])
  pltpu.sync_copy(sq_hbm, sq_v)
  pltpu.sync_copy(sk_hbm, sk_v)
  plsc.subcore_barrier()

  bufs = [(dstet_v0, srk_v0, dst2_v0, ex_v0, seml0, sems0),
          (dstet_v1, srk_v1, dst2_v1, ex_v1, seml1, sems1)]

  def loadf(g, buf):
    dstet_v, srk_v, dst2_v, _, seml, _ = bufs[buf]
    rbase = wid * (EP // SUB) + g * NRC
    pltpu.async_copy(dstet2_hbm.at[pl.ds(rbase, NRC)], dstet_v, seml)
    pltpu.async_copy(srk2_hbm.at[pl.ds(rbase, NRC)], srk_v, seml)
    pltpu.async_copy(dst2_hbm.at[pl.ds(rbase, NRC)], dst2_v, seml)

  def compute(g, buf):
    dstet_v, srk_v, dst2_v, ex_v, seml, sems = bufs[buf]
    rbase = wid * (EP // SUB)
    pltpu.make_async_copy(dstet2_hbm.at[pl.ds(rbase, NRC)], dstet_v,
                          seml).wait()
    pltpu.make_async_copy(srk2_hbm.at[pl.ds(rbase, NRC)], srk_v,
                          seml).wait()
    pltpu.make_async_copy(dst2_hbm.at[pl.ds(rbase, NRC)], dst2_v,
                          seml).wait()
    for v in range(NVC):
      row, col = v // (SUB // 16), (v % (SUB // 16)) * 16
      csl = pl.ds(col, 16)
      qi = plsc.load_gather(sq_v, [dstet_v[row, csl]])
      kj = plsc.load_gather(sk_v, [srk_v[row, csl]])
      a = qi + kj
      a = jnp.maximum(a, a * NEG)
      ex_v[pl.ds(v * 16, 16)] = jnp.exp(a)
    del sems
    for j in range(NRC):
      pltpu.sync_copy(ex_v.at[pl.ds(j * SUB, SUB)],
                      den_sh.at[dst2_v.at[j]], add=True)
    base = wid * EP + g * C
    pltpu.sync_copy(ex_v.at[pl.ds(0, C)], ex_hbm.at[pl.ds(base, C)])

  loadf(0, 0)

  def pair(h, _):
    loadf(2 * h + 1, 1)
    compute(2 * h, 0)
    loadf(2 * h + 2, 0)
    compute(2 * h + 1, 1)
    return 0
  lax.fori_loop(0, (NCH - 1) // 2, pair, 0)
  compute(NCH - 1, 0)
  plsc.subcore_barrier()
  pltpu.sync_copy(den_sh.at[pl.ds(sid * PT, PT)],
                  den_hbm.at[cid, pl.ds(sid * PT, PT)])


_sc_edge_a = pl.kernel(
    _sc_a_body,
    out_type=(jax.ShapeDtypeStruct((E,), F32),
              jax.ShapeDtypeStruct((NCORE, NPAD), F32)),
    mesh=_MESH,
    compiler_params=_SC_PARAMS,
    scratch_types=[
        pltpu.VMEM((N * R,), F32),
        pltpu.VMEM((N * R,), F32),
        pltpu.VMEM((NRC, SUB), I32),
        pltpu.VMEM((NRC, SUB), I32),
        pltpu.VMEM((NRC, SUB), I32),
        pltpu.VMEM((NRC, SUB), I32),
        pltpu.VMEM((NRC, SUB), I32),
        pltpu.VMEM((NRC, SUB), I32),
        pltpu.VMEM((PT,), F32),
        pltpu.VMEM((PT,), F32),
        pltpu.VMEM_SHARED((NPAD,), F32),
        pltpu.SemaphoreType.DMA,
        pltpu.SemaphoreType.DMA,
        pltpu.SemaphoreType.DMA,
        pltpu.SemaphoreType.DMA,
    ])


GB = 2000
NGB = EP // GB
NCG = GB // C
NRG = GB // SUB


def _sc_b_body(dout, xrel_hbm, ex_hbm, den_hbm, srk2_hbm, dst2_hbm,
               outp_hbm,
               den0_v, den1_v, srk_v, dst2_v, ex_v,
               w_v0, w_v1, rows_v0, rows_v1, out_sh,
               semg0, semg1, sems0, sems1):
  cid = lax.axis_index("c")
  sid = lax.axis_index("s")
  wid = cid * NTILE + sid
  ncb = dout // 16
  nrt = NPAD // NTILE

  def zr(e, _):
    for cc in range(ncb):
      rows_v0[e, pl.ds(cc * 16, 16)] = jnp.zeros((16,), F32)
    return 0
  lax.fori_loop(0, C, zr, 0)
  pltpu.sync_copy(rows_v0, out_sh.at[pl.ds(sid * nrt, C)])
  pltpu.sync_copy(rows_v0.at[pl.ds(0, nrt - C)],
                  out_sh.at[pl.ds(sid * nrt + C, nrt - C)])
  pltpu.sync_copy(den_hbm.at[0], den0_v)
  pltpu.sync_copy(den_hbm.at[1], den1_v)

  def dc(i, _):
    sl = pl.ds(i * 16, 16)
    den0_v[sl] = den0_v[sl] + den1_v[sl]
    return 0
  lax.fori_loop(0, NPAD // 16, dc, 0)
  plsc.subcore_barrier()

  bufs = [(w_v0, rows_v0, semg0, sems0), (w_v1, rows_v1, semg1, sems1)]

  def drain_scat(buf):
    _, rows_v, _, sems = bufs[buf]
    pltpu.make_async_copy(xrel_hbm.at[pl.ds(0, C)], rows_v, sems).wait()

  def fire(c, buf, drain):
    w_v, rows_v, semg, _ = bufs[buf]
    if drain:
      drain_scat(buf)
    for v in range(C // 16):
      gv = c * (C // 16) + v
      row, col = gv // (SUB // 16), (gv % (SUB // 16)) * 16
      d = dst2_v[row, pl.ds(col, 16)]
      den = plsc.load_gather(den0_v, [d])
      w_v[pl.ds(v * 16, 16)] = ex_v[pl.ds(c * C + v * 16, 16)] / (den + 1e-16)
    for j in range(NSB):
      pltpu.async_copy(xrel_hbm.at[srk_v.at[c * NSB + j]],
                       rows_v.at[pl.ds(j * SUB, SUB)], semg)

  def process(c, buf):
    w_v, rows_v, semg, sems = bufs[buf]
    pltpu.make_async_copy(xrel_hbm.at[pl.ds(0, C)], rows_v, semg).wait()

    def scale(b, _):
      wv = w_v[pl.ds(b * 16, 16)]
      for j in range(16):
        e = b * 16 + j
        wj = jnp.broadcast_to(wv[j], (16,))
        for cc in range(ncb):
          sl2 = pl.ds(cc * 16, 16)
          rows_v[e, sl2] = rows_v[e, sl2] * wj
      return 0
    lax.fori_loop(0, C // 16, scale, 0)
    for j in range(NSB):
      pltpu.async_copy(rows_v.at[pl.ds(j * SUB, SUB)],
                       out_sh.at[dst2_v.at[c * NSB + j]], sems, add=True)

  def group(g, _):
    rbase = wid * (EP // SUB) + g * NRG
    pltpu.sync_copy(srk2_hbm.at[pl.ds(rbase, NRG)], srk_v)
    pltpu.sync_copy(dst2_hbm.at[pl.ds(rbase, NRG)], dst2_v)
    pltpu.sync_copy(ex_hbm.at[pl.ds(wid * EP + g * GB, GB)], ex_v)
    fire(0, 0, drain=False)
    for c in range(1, NCG):
      fire(c, c % 2, drain=c >= 2)
      process(c - 1, (c - 1) % 2)
    process(NCG - 1, (NCG - 1) % 2)
    drain_scat((NCG - 1) % 2)
    drain_scat((NCG - 2) % 2)
    return 0
  lax.fori_loop(0, NGB, group, 0)

  plsc.subcore_barrier()
  pltpu.sync_copy(out_sh.at[pl.ds(sid * nrt, nrt)],
                  outp_hbm.at[cid, pl.ds(sid * nrt, nrt)])


def _make_sc_b(dout):
  return pl.kernel(
      functools.partial(_sc_b_body, dout),
      out_type=jax.ShapeDtypeStruct((NCORE, NPAD, dout), F32),
      mesh=_MESH,
      compiler_params=_SC_PARAMS,
      scratch_types=[
          pltpu.VMEM((NPAD,), F32),
          pltpu.VMEM((NPAD,), F32),
          pltpu.VMEM((NRG, SUB), I32),
          pltpu.VMEM((NRG, SUB), I32),
          pltpu.VMEM((GB,), F32),
          pltpu.VMEM((C,), F32),
          pltpu.VMEM((C,), F32),
          pltpu.VMEM((C, dout), F32),
          pltpu.VMEM((C, dout), F32),
          pltpu.VMEM_SHARED((NPAD, dout), F32),
          pltpu.SemaphoreType.DMA,
          pltpu.SemaphoreType.DMA,
          pltpu.SemaphoreType.DMA,
          pltpu.SemaphoreType.DMA,
      ])


_sc_edge_b64 = _make_sc_b(H)
_sc_edge_b16 = _make_sc_b(DP3)


def kernel(x, edge_index, edge_type, w1, q1, k1, b1, w2, q2, k2, b2,
           w3, q3, k3, b3):
  src = edge_index[0].astype(I32)
  dst = edge_index[1].astype(I32)
  et = edge_type.astype(I32)
  srk2 = (src * R + et).reshape(E // SUB, SUB)
  dstet2 = (dst * R + et).reshape(E // SUB, SUB)
  dst2 = dst.reshape(E // SUB, SUB)

  def edge_phase(xrel, sq, sk, sc_b, dpad):
    ex, den = _sc_edge_a(sq.reshape(N * R), sk.reshape(N * R),
                         dstet2, srk2, dst2)
    return sc_b(xrel.reshape(N * R, dpad), ex, den, srk2, dst2)

  xrel, sq, sk = _tc_layer(D1, H, H, True)(x, w1, q1, k1)
  outp = edge_phase(xrel, sq, sk, _sc_edge_b64, H)

  xrel, sq, sk = _tc_layer(H, H, H, False)(
      outp, b1.reshape(1, H), w2, q2, k2)
  outp = edge_phase(xrel, sq, sk, _sc_edge_b64, H)

  xrel, sq, sk = _tc_layer(H, NCLS, DP3, False)(
      outp, b2.reshape(1, H), w3, q3, k3)
  outp = edge_phase(xrel, sq, sk, _sc_edge_b16, DP3)

  return _tc_final(outp, b3.reshape(1, NCLS))
